# baseline (speedup 1.0000x reference)
import functools

import jax
import jax.numpy as jnp
from jax import lax
from jax.experimental import pallas as pl
from jax.experimental.pallas import tpu as pltpu
from jax.experimental.pallas import tpu_sc as plsc

NC = 2
NS = 16
NW = NC * NS
LANES = 16


def _sc_gather_products(users_id, items_id,
                        user_emb_inv, item_emb_inv,
                        user_emb_env, item_emb_env,
                        user_bias_inv, item_bias_inv,
                        user_bias_env, item_bias_env):
    B = users_id.shape[0]
    D = user_emb_inv.shape[1]
    b_per_w = B // NW
    CH = 128
    n_ch = b_per_w // CH

    ids2d_u = users_id.reshape(B // CH, CH)
    ids2d_i = items_id.reshape(B // CH, CH)

    mesh = plsc.VectorSubcoreMesh(core_axis_name="c", subcore_axis_name="s",
                                  num_cores=NC, num_subcores=NS)

    f32 = jnp.float32
    out_type = (
        jax.ShapeDtypeStruct((B, D), f32),
        jax.ShapeDtypeStruct((B, D), f32),
        jax.ShapeDtypeStruct((B,), f32),
        jax.ShapeDtypeStruct((B,), f32),
    )

    @functools.partial(
        pl.kernel,
        out_type=out_type,
        mesh=mesh,
        compiler_params=pltpu.CompilerParams(use_tc_tiling_on_sc=False),
        scratch_types=[
            pltpu.VMEM((n_ch, CH), jnp.int32),
            pltpu.VMEM((n_ch, CH), jnp.int32),
            pltpu.VMEM((b_per_w, D), f32),
            pltpu.VMEM((b_per_w, D), f32),
            pltpu.VMEM((b_per_w, D), f32),
            pltpu.VMEM((b_per_w, D), f32),
            pltpu.VMEM((b_per_w,), f32),
            pltpu.VMEM((b_per_w,), f32),
            pltpu.VMEM((b_per_w,), f32),
            pltpu.VMEM((b_per_w,), f32),
            pltpu.SemaphoreType.DMA,
        ],
    )
    def sc_kernel(uid_hbm, iid_hbm, ue_hbm, ie_hbm, uenvv_hbm, ienvv_hbm,
                  ub_hbm, ib_hbm, ube_hbm, ibe_hbm,
                  invp_out, envuv_out, binv_out, benv_out,
                  uidx_v, iidx_v, u_v, i_v, ue_v, ie_v,
                  ub_v, ib_v, ube_v, ibe_v, sem):
        wid = lax.axis_index("s") * NC + lax.axis_index("c")
        base = wid * b_per_w
        ch0 = wid * n_ch

        pltpu.sync_copy(uid_hbm.at[pl.ds(ch0, n_ch)], uidx_v)
        pltpu.sync_copy(iid_hbm.at[pl.ds(ch0, n_ch)], iidx_v)

        copies = []
        for j in range(n_ch):
            rows = pl.ds(j * CH, CH)
            uix = uidx_v.at[j]
            iix = iidx_v.at[j]
            copies.append(pltpu.async_copy(ue_hbm.at[uix], u_v.at[rows], sem))
            copies.append(pltpu.async_copy(ie_hbm.at[iix], i_v.at[rows], sem))
            copies.append(pltpu.async_copy(uenvv_hbm.at[uix], ue_v.at[rows], sem))
            copies.append(pltpu.async_copy(ienvv_hbm.at[iix], ie_v.at[rows], sem))
            copies.append(pltpu.async_copy(ub_hbm.at[uix], ub_v.at[rows], sem))
            copies.append(pltpu.async_copy(ib_hbm.at[iix], ib_v.at[rows], sem))
            copies.append(pltpu.async_copy(ube_hbm.at[uix], ube_v.at[rows], sem))
            copies.append(pltpu.async_copy(ibe_hbm.at[iix], ibe_v.at[rows], sem))
        for c in copies:
            c.wait()

        @pl.loop(0, b_per_w)
        def _(r):
            for h in range(D // LANES):
                sl = pl.ds(h * LANES, LANES)
                u_v[r, sl] = u_v[r, sl] * i_v[r, sl]
                ue_v[r, sl] = ue_v[r, sl] * ie_v[r, sl]

        @pl.loop(0, b_per_w, step=LANES)
        def _(r):
            sl = pl.ds(r, LANES)
            ub_v[sl] = ub_v[sl] + ib_v[sl]
            ube_v[sl] = ube_v[sl] + ibe_v[sl]

        out_rows = pl.ds(base, b_per_w)
        pltpu.sync_copy(u_v, invp_out.at[out_rows])
        pltpu.sync_copy(ue_v, envuv_out.at[out_rows])
        pltpu.sync_copy(ub_v, binv_out.at[out_rows])
        pltpu.sync_copy(ube_v, benv_out.at[out_rows])

    return sc_kernel(ids2d_u, ids2d_i, user_emb_inv, item_emb_inv,
                     user_emb_env, item_emb_env,
                     user_bias_inv.reshape(-1), item_bias_inv.reshape(-1),
                     user_bias_env.reshape(-1), item_bias_env.reshape(-1))


def _tc_tail_body(invp_ref, envuv_ref, binv_ref, benv_ref, eid_ref,
                  envemb_ref, envb_ref, wt_ref, cb_ref,
                  inv_out, env_out, lsm_out):
    NE = envemb_ref.shape[0]
    invp = invp_ref[...]
    envuv = envuv_ref[...]
    eid = eid_ref[...]
    onehot = (eid == lax.broadcasted_iota(jnp.int32, (1, NE), 1)
              ).astype(jnp.float32)

    envrow = jnp.zeros_like(envuv)
    for e in range(NE):
        envrow = envrow + onehot[:, e:e + 1] * envemb_ref[e, :][None, :]
    env_b = jnp.sum(onehot * envb_ref[...], axis=1)

    inv_score = jnp.sum(invp, axis=1) + binv_ref[...]
    env_mid = jnp.sum(envuv * envrow, axis=1) + benv_ref[...] + env_b
    env_score = inv_score + env_mid

    logits = jax.lax.dot_general(
        invp, wt_ref[...], (((1,), (0,)), ((), ())),
        precision=lax.Precision.HIGHEST,
        preferred_element_type=jnp.float32) + cb_ref[...]
    m = jnp.max(logits, axis=1, keepdims=True)
    s = logits - m
    lse = jnp.log(jnp.sum(jnp.exp(s), axis=1, keepdims=True))

    inv_out[...] = inv_score
    env_out[...] = env_score
    lsm_out[...] = s - lse


def _tc_tail(inv_pref, envuv, bias_inv, bias_env, envs_id,
             env_emb, env_bias, cls_W, cls_b):
    B, D = inv_pref.shape
    NE = env_emb.shape[0]
    BLK = 2048
    grid = (B // BLK,)
    f32 = jnp.float32

    return pl.pallas_call(
        _tc_tail_body,
        grid=grid,
        in_specs=[
            pl.BlockSpec((BLK, D), lambda i: (i, 0)),
            pl.BlockSpec((BLK, D), lambda i: (i, 0)),
            pl.BlockSpec((BLK,), lambda i: (i,)),
            pl.BlockSpec((BLK,), lambda i: (i,)),
            pl.BlockSpec((BLK, 1), lambda i: (i, 0)),
            pl.BlockSpec((NE, D), lambda i: (0, 0)),
            pl.BlockSpec((1, NE), lambda i: (0, 0)),
            pl.BlockSpec((D, NE), lambda i: (0, 0)),
            pl.BlockSpec((1, NE), lambda i: (0, 0)),
        ],
        out_specs=[
            pl.BlockSpec((BLK,), lambda i: (i,)),
            pl.BlockSpec((BLK,), lambda i: (i,)),
            pl.BlockSpec((BLK, NE), lambda i: (i, 0)),
        ],
        out_shape=[
            jax.ShapeDtypeStruct((B,), f32),
            jax.ShapeDtypeStruct((B,), f32),
            jax.ShapeDtypeStruct((B, NE), f32),
        ],
    )(inv_pref, envuv, bias_inv, bias_env, envs_id.reshape(B, 1),
      env_emb, env_bias.reshape(1, NE), cls_W.T, cls_b.reshape(1, NE))


def kernel(users_id, items_id, envs_id, alpha,
           user_emb_inv, user_bias_inv, item_emb_inv, item_bias_inv,
           user_emb_env, user_bias_env, item_emb_env, item_bias_env,
           env_emb, env_bias, cls_W, cls_b):
    del alpha
    inv_pref, envuv, bias_inv, bias_env = _sc_gather_products(
        users_id, items_id, user_emb_inv, item_emb_inv,
        user_emb_env, item_emb_env,
        user_bias_inv, item_bias_inv, user_bias_env, item_bias_env)
    inv_score, env_score, env_outputs = _tc_tail(
        inv_pref, envuv, bias_inv, bias_env, envs_id,
        env_emb, env_bias, cls_W, cls_b)
    return inv_score, env_score, env_outputs


# speedup vs baseline: 1.0007x; 1.0007x over previous
import functools

import jax
import jax.numpy as jnp
from jax import lax
from jax.experimental import pallas as pl
from jax.experimental.pallas import tpu as pltpu
from jax.experimental.pallas import tpu_sc as plsc

NC = 2
NS = 16
NW = NC * NS
LANES = 16


def _sc_gather_products(users_id, items_id,
                        user_emb_inv, item_emb_inv,
                        user_emb_env, item_emb_env,
                        user_bias_inv, item_bias_inv,
                        user_bias_env, item_bias_env):
    B = users_id.shape[0]
    D = user_emb_inv.shape[1]
    b_per_w = B // NW
    CH = 128
    n_ch = b_per_w // CH

    ids2d_u = users_id.reshape(B // CH, CH)
    ids2d_i = items_id.reshape(B // CH, CH)

    mesh = plsc.VectorSubcoreMesh(core_axis_name="c", subcore_axis_name="s",
                                  num_cores=NC, num_subcores=NS)

    f32 = jnp.float32
    out_type = (
        jax.ShapeDtypeStruct((B, D), f32),
        jax.ShapeDtypeStruct((B, D), f32),
        jax.ShapeDtypeStruct((B,), f32),
        jax.ShapeDtypeStruct((B,), f32),
    )

    @functools.partial(
        pl.kernel,
        out_type=out_type,
        mesh=mesh,
        compiler_params=pltpu.CompilerParams(use_tc_tiling_on_sc=False,
                                             needs_layout_passes=False),
        scratch_types=[
            pltpu.VMEM((n_ch, CH), jnp.int32),
            pltpu.VMEM((n_ch, CH), jnp.int32),
            pltpu.VMEM((b_per_w, D), f32),
            pltpu.VMEM((b_per_w, D), f32),
            pltpu.VMEM((b_per_w, D), f32),
            pltpu.VMEM((b_per_w, D), f32),
            pltpu.VMEM((n_ch, CH), jnp.int32),
            pltpu.VMEM((n_ch, CH), jnp.int32),
            pltpu.VMEM((b_per_w, LANES), f32),
            pltpu.VMEM((b_per_w, LANES), f32),
            pltpu.VMEM((b_per_w, LANES), f32),
            pltpu.VMEM((b_per_w, LANES), f32),
            pltpu.VMEM((b_per_w,), f32),
            pltpu.VMEM((b_per_w,), f32),
            pltpu.SemaphoreType.DMA,
        ],
    )
    def sc_kernel(uid_hbm, iid_hbm, ue_hbm, ie_hbm, uenvv_hbm, ienvv_hbm,
                  ub2, ib2, ube2, ibe2,
                  invp_out, envuv_out, binv_out, benv_out,
                  uidx_v, iidx_v, u_v, i_v, ue_v, ie_v, uridx_v, iridx_v,
                  ubr_v, ibr_v, uber_v, iber_v, binv_v, benv_v, sem):
        wid = lax.axis_index("s") * NC + lax.axis_index("c")
        base = wid * b_per_w
        ch0 = wid * n_ch

        pltpu.sync_copy(uid_hbm.at[pl.ds(ch0, n_ch)], uidx_v)
        pltpu.sync_copy(iid_hbm.at[pl.ds(ch0, n_ch)], iidx_v)

        for j in range(n_ch):
            for o in range(0, CH, LANES):
                sl = pl.ds(o, LANES)
                uridx_v[j, sl] = lax.shift_right_logical(uidx_v[j, sl], 4)
                iridx_v[j, sl] = lax.shift_right_logical(iidx_v[j, sl], 4)

        copies = []
        for j in range(n_ch):
            rows = pl.ds(j * CH, CH)
            uix = uidx_v.at[j]
            iix = iidx_v.at[j]
            urix = uridx_v.at[j]
            irix = iridx_v.at[j]
            copies.append(pltpu.async_copy(ue_hbm.at[uix], u_v.at[rows], sem))
            copies.append(pltpu.async_copy(ie_hbm.at[iix], i_v.at[rows], sem))
            copies.append(pltpu.async_copy(uenvv_hbm.at[uix], ue_v.at[rows], sem))
            copies.append(pltpu.async_copy(ienvv_hbm.at[iix], ie_v.at[rows], sem))
            copies.append(pltpu.async_copy(ub2.at[urix], ubr_v.at[rows], sem))
            copies.append(pltpu.async_copy(ib2.at[irix], ibr_v.at[rows], sem))
            copies.append(pltpu.async_copy(ube2.at[urix], uber_v.at[rows], sem))
            copies.append(pltpu.async_copy(ibe2.at[irix], iber_v.at[rows], sem))
        for c in copies:
            c.wait()

        @pl.loop(0, b_per_w)
        def _(r):
            for h in range(D // LANES):
                sl = pl.ds(h * LANES, LANES)
                u_v[r, sl] = u_v[r, sl] * i_v[r, sl]
                ue_v[r, sl] = ue_v[r, sl] * ie_v[r, sl]

        iota16 = lax.iota(jnp.int32, LANES)
        for k in range(0, b_per_w, LANES):
            j, o = k // CH, k % CH
            sl = pl.ds(o, LANES)
            rows16 = iota16 + k
            ucol = lax.bitwise_and(uidx_v[j, sl], LANES - 1)
            icol = lax.bitwise_and(iidx_v[j, sl], LANES - 1)
            binv_v[pl.ds(k, LANES)] = (
                plsc.load_gather(ubr_v, [rows16, ucol])
                + plsc.load_gather(ibr_v, [rows16, icol]))
            benv_v[pl.ds(k, LANES)] = (
                plsc.load_gather(uber_v, [rows16, ucol])
                + plsc.load_gather(iber_v, [rows16, icol]))

        out_rows = pl.ds(base, b_per_w)
        pltpu.sync_copy(u_v, invp_out.at[out_rows])
        pltpu.sync_copy(ue_v, envuv_out.at[out_rows])
        pltpu.sync_copy(binv_v, binv_out.at[out_rows])
        pltpu.sync_copy(benv_v, benv_out.at[out_rows])

    return sc_kernel(ids2d_u, ids2d_i, user_emb_inv, item_emb_inv,
                     user_emb_env, item_emb_env,
                     user_bias_inv.reshape(-1, LANES),
                     item_bias_inv.reshape(-1, LANES),
                     user_bias_env.reshape(-1, LANES),
                     item_bias_env.reshape(-1, LANES))


def _tc_tail_body(invp_ref, envuv_ref, binv_ref, benv_ref,
                  eid_ref, envemb_ref, envb_ref, wt_ref, cb_ref,
                  inv_out, env_out, lsm_out):
    NE = envemb_ref.shape[0]
    invp = invp_ref[...]
    envuv = envuv_ref[...]
    eid = eid_ref[...]
    onehot = (eid == lax.broadcasted_iota(jnp.int32, (1, NE), 1)
              ).astype(jnp.float32)

    envrow = jnp.zeros_like(envuv)
    for e in range(NE):
        envrow = envrow + onehot[:, e:e + 1] * envemb_ref[e, :][None, :]
    env_b = jnp.sum(onehot * envb_ref[...], axis=1)

    inv_score = jnp.sum(invp, axis=1) + binv_ref[...]
    env_mid = jnp.sum(envuv * envrow, axis=1) + benv_ref[...] + env_b
    env_score = inv_score + env_mid

    logits = jax.lax.dot_general(
        invp, wt_ref[...], (((1,), (0,)), ((), ())),
        precision=lax.Precision.HIGHEST,
        preferred_element_type=jnp.float32) + cb_ref[...]
    m = jnp.max(logits, axis=1, keepdims=True)
    s = logits - m
    lse = jnp.log(jnp.sum(jnp.exp(s), axis=1, keepdims=True))

    inv_out[...] = inv_score
    env_out[...] = env_score
    lsm_out[...] = s - lse


def _tc_tail(inv_pref, envuv, bias_inv, bias_env, envs_id,
             env_emb, env_bias, cls_W, cls_b):
    B, D = inv_pref.shape
    NE = env_emb.shape[0]
    BLK = 2048
    grid = (B // BLK,)
    f32 = jnp.float32

    return pl.pallas_call(
        _tc_tail_body,
        grid=grid,
        in_specs=[
            pl.BlockSpec((BLK, D), lambda i: (i, 0)),
            pl.BlockSpec((BLK, D), lambda i: (i, 0)),
            pl.BlockSpec((BLK,), lambda i: (i,)),
            pl.BlockSpec((BLK,), lambda i: (i,)),
            pl.BlockSpec((BLK, 1), lambda i: (i, 0)),
            pl.BlockSpec((NE, D), lambda i: (0, 0)),
            pl.BlockSpec((1, NE), lambda i: (0, 0)),
            pl.BlockSpec((D, NE), lambda i: (0, 0)),
            pl.BlockSpec((1, NE), lambda i: (0, 0)),
        ],
        out_specs=[
            pl.BlockSpec((BLK,), lambda i: (i,)),
            pl.BlockSpec((BLK,), lambda i: (i,)),
            pl.BlockSpec((BLK, NE), lambda i: (i, 0)),
        ],
        out_shape=[
            jax.ShapeDtypeStruct((B,), f32),
            jax.ShapeDtypeStruct((B,), f32),
            jax.ShapeDtypeStruct((B, NE), f32),
        ],
    )(inv_pref, envuv, bias_inv, bias_env, envs_id.reshape(B, 1),
      env_emb, env_bias.reshape(1, NE), cls_W.T, cls_b.reshape(1, NE))


def kernel(users_id, items_id, envs_id, alpha,
           user_emb_inv, user_bias_inv, item_emb_inv, item_bias_inv,
           user_emb_env, user_bias_env, item_emb_env, item_bias_env,
           env_emb, env_bias, cls_W, cls_b):
    del alpha
    inv_pref, envuv, bias_inv, bias_env = _sc_gather_products(
        users_id, items_id, user_emb_inv, item_emb_inv,
        user_emb_env, item_emb_env,
        user_bias_inv, item_bias_inv, user_bias_env, item_bias_env)
    inv_score, env_score, env_outputs = _tc_tail(
        inv_pref, envuv, bias_inv, bias_env, envs_id,
        env_emb, env_bias, cls_W, cls_b)
    return inv_score, env_score, env_outputs
